# Initial kernel scaffold; baseline (speedup 1.0000x reference)
import jax
import jax.numpy as jnp
from jax.experimental import pallas as pl


def kernel(x, edge_index, edge_attr, Wl1, bl1, Wr1, Wl2, bl2, Wr2, Wc1, bc1, Wc2, bc2, Wc3, bc3):
    raise NotImplementedError("write your pallas kernel here")


# speedup vs baseline: 4.9164x; 4.9164x over previous
import functools

import jax
import jax.numpy as jnp
from jax import lax
from jax.experimental import pallas as pl
from jax.experimental.pallas import tpu as pltpu
from jax.experimental.pallas import tpu_sc as plsc

N = 10000
E = 320000
D = 128
DE = 16
H = 128

_NC = 2
_NS = 16
_NW = _NC * _NS
_NPAD = 10112
_ROWS_PER_SC = _NPAD // _NS

_CHUNK = 128


def _fill_rows(ref, nrows, ncols, val):
    v = jnp.full((16,), val, jnp.float32)
    nc = ncols // 16

    def body(r, c):
        for j in range(nc):
            ref[r, pl.ds(j * 16, 16)] = v
        return c

    lax.fori_loop(0, nrows, body, 0)


def _make_seg_sum(with_deg):
    n_super = E // _CHUNK
    base_cnt = n_super // _NW
    extra = n_super - base_cnt * _NW

    out_type = [jax.ShapeDtypeStruct((_NC, _NPAD, D), jnp.float32)]
    if with_deg:
        out_type.append(jax.ShapeDtypeStruct((_NC, _NPAD, D), jnp.float32))
    scratch = [
        pltpu.VMEM((1, _CHUNK), jnp.int32),
        pltpu.VMEM((1, _CHUNK), jnp.int32),
        pltpu.VMEM((_CHUNK, D), jnp.float32),
        pltpu.VMEM_SHARED((_NPAD, D), jnp.float32),
        pltpu.SemaphoreType.DMA,
    ]

    def body(table, src2d, dst2d, *refs):
        if with_deg:
            aggp, degp, sidx, didx, rows, agg_s, sem = refs
        else:
            aggp, sidx, didx, rows, agg_s, sem = refs
        cid = lax.axis_index("c")
        sid = lax.axis_index("s")
        wid = cid * _NS + sid
        r0 = sid * _ROWS_PER_SC
        rem = _ROWS_PER_SC % _CHUNK
        start = base_cnt * wid + jnp.minimum(wid, extra)
        cnt = base_cnt + jnp.where(wid < extra, 1, 0)

        def zero_my_slice():
            _fill_rows(rows, _CHUNK, D, 0.0)
            for z in range(_ROWS_PER_SC // _CHUNK):
                pltpu.sync_copy(rows, agg_s.at[pl.ds(r0 + z * _CHUNK, _CHUNK)])
            pltpu.sync_copy(rows.at[pl.ds(0, rem)],
                            agg_s.at[pl.ds(r0 + _ROWS_PER_SC - rem, rem)])

        zero_my_slice()
        plsc.subcore_barrier()

        def step(b, c):
            sb = start + b
            pltpu.sync_copy(src2d.at[pl.ds(sb, 1)], sidx)
            pltpu.sync_copy(dst2d.at[pl.ds(sb, 1)], didx)
            pltpu.async_copy(table.at[sidx.at[0]], rows, sem).wait()
            pltpu.sync_copy(rows, agg_s.at[didx.at[0]], add=True)
            return c

        lax.fori_loop(0, cnt, step, 0)
        plsc.subcore_barrier()

        pltpu.sync_copy(agg_s.at[pl.ds(r0, _ROWS_PER_SC)],
                        aggp.at[cid, pl.ds(r0, _ROWS_PER_SC)])

        if with_deg:
            zero_my_slice()
            _fill_rows(rows, _CHUNK, D, 1.0)
            plsc.subcore_barrier()

            def dstep(b, c):
                sb = start + b
                pltpu.sync_copy(dst2d.at[pl.ds(sb, 1)], didx)
                pltpu.sync_copy(rows, agg_s.at[didx.at[0]], add=True)
                return c

            lax.fori_loop(0, cnt, dstep, 0)
            plsc.subcore_barrier()
            pltpu.sync_copy(agg_s.at[pl.ds(r0, _ROWS_PER_SC)],
                            degp.at[cid, pl.ds(r0, _ROWS_PER_SC)])

    mesh = plsc.VectorSubcoreMesh(core_axis_name="c", subcore_axis_name="s")
    return pl.kernel(body, out_type=out_type, mesh=mesh,
                     scratch_types=scratch,
                     name="sc_seg_sum_deg" if with_deg else "sc_seg_sum")


_seg_sum_deg = _make_seg_sum(True)
_seg_sum = _make_seg_sum(False)


def _make_edge_gather():
    n_blk = E // 256
    base_cnt = n_blk // _NW
    extra = n_blk - base_cnt * _NW

    out_type = [jax.ShapeDtypeStruct((E, D), jnp.float32),
                jax.ShapeDtypeStruct((E, D), jnp.float32)]
    scratch = [
        pltpu.VMEM((2, _CHUNK), jnp.int32),
        pltpu.VMEM((2, _CHUNK), jnp.int32),
        pltpu.VMEM((256, D), jnp.float32),
        pltpu.VMEM((256, D), jnp.float32),
        pltpu.SemaphoreType.DMA,
    ]

    def body(ps, pd, src2d, dst2d, a_out, b_out, sidx, didx, ra, rb, sem):
        cid = lax.axis_index("c")
        sid = lax.axis_index("s")
        wid = cid * _NS + sid
        start = base_cnt * wid + jnp.minimum(wid, extra)
        cnt = base_cnt + jnp.where(wid < extra, 1, 0)

        def step(b, c):
            sb = start + b
            crow = sb * 2
            pltpu.sync_copy(src2d.at[pl.ds(crow, 2)], sidx)
            pltpu.sync_copy(dst2d.at[pl.ds(crow, 2)], didx)
            descs = []
            for j in range(2):
                descs.append(pltpu.async_copy(
                    ps.at[sidx.at[j]],
                    ra.at[pl.ds(j * _CHUNK, _CHUNK)], sem))
                descs.append(pltpu.async_copy(
                    pd.at[didx.at[j]],
                    rb.at[pl.ds(j * _CHUNK, _CHUNK)], sem))
            for d in descs:
                d.wait()
            pltpu.sync_copy(ra, a_out.at[pl.ds(sb * 256, 256)])
            pltpu.sync_copy(rb, b_out.at[pl.ds(sb * 256, 256)])
            return c

        lax.fori_loop(0, cnt, step, 0)

    mesh = plsc.VectorSubcoreMesh(core_axis_name="c", subcore_axis_name="s")
    return pl.kernel(body, out_type=out_type, mesh=mesh,
                     scratch_types=scratch, name="sc_edge_gather")


_edge_gather = _make_edge_gather()


def _dotT(a, w):
    return lax.dot_general(a, w, (((1,), (1,)), ((), ())),
                           preferred_element_type=jnp.float32)


def _sage_body(aggp, degp, x, wl, bl, wr, out):
    a = aggp[...]
    dp = degp[...]
    deg = dp[0, :N, 0:1] + dp[1, :N, 0:1]
    rdeg = 1.0 / jnp.maximum(deg, 1.0)
    mean = (a[0, :N] + a[1, :N]) * rdeg
    h = _dotT(mean, wl[...]) + bl[...] + _dotT(x[...], wr[...])
    out[...] = jnp.maximum(h, 0.0)


_tc_layer1 = pl.pallas_call(
    _sage_body,
    out_shape=jax.ShapeDtypeStruct((N, D), jnp.float32),
)


def _sage2_body(aggp2, degp, h1, wl, bl, wr, wc1s, wc1d, ps_out, pd_out):
    a = aggp2[...]
    dp = degp[...]
    deg = dp[0, :N, 0:1] + dp[1, :N, 0:1]
    rdeg = 1.0 / jnp.maximum(deg, 1.0)
    mean = (a[0, :N] + a[1, :N]) * rdeg
    h = _dotT(mean, wl[...]) + bl[...] + _dotT(h1[...], wr[...])
    h2 = jnp.maximum(h, 0.0)
    ps_out[...] = _dotT(h2, wc1s[...])
    pd_out[...] = _dotT(h2, wc1d[...])


_tc_layer2 = pl.pallas_call(
    _sage2_body,
    out_shape=[jax.ShapeDtypeStruct((N, D), jnp.float32),
               jax.ShapeDtypeStruct((N, D), jnp.float32)],
)


_EBLK = 16000


def _edge_mlp_body(a, b, ea, wc1e, bc1, wc2, bc2, wc3, bc3, out):
    z1 = a[...] + b[...] + _dotT(ea[...], wc1e[...]) + bc1[...]
    z1 = jnp.maximum(z1, 0.0)
    z2 = jnp.maximum(_dotT(z1, wc2[...]) + bc2[...], 0.0)
    lo = lax.dot_general(wc3[...], z2, (((1,), (1,)), ((), ())),
                         preferred_element_type=jnp.float32) + bc3[...]
    out[...] = 1.0 / (1.0 + jnp.exp(-lo))


_tc_edge_mlp = pl.pallas_call(
    _edge_mlp_body,
    grid=(E // _EBLK,),
    in_specs=[
        pl.BlockSpec((_EBLK, D), lambda i: (i, 0)),
        pl.BlockSpec((_EBLK, D), lambda i: (i, 0)),
        pl.BlockSpec((_EBLK, DE), lambda i: (i, 0)),
        pl.BlockSpec((H, DE), lambda i: (0, 0)),
        pl.BlockSpec((1, H), lambda i: (0, 0)),
        pl.BlockSpec((64, H), lambda i: (0, 0)),
        pl.BlockSpec((1, 64), lambda i: (0, 0)),
        pl.BlockSpec((1, 64), lambda i: (0, 0)),
        pl.BlockSpec((1, 1), lambda i: (0, 0)),
    ],
    out_specs=pl.BlockSpec((1, _EBLK), lambda i: (0, i)),
    out_shape=jax.ShapeDtypeStruct((1, E), jnp.float32),
)


def kernel(x, edge_index, edge_attr, Wl1, bl1, Wr1, Wl2, bl2, Wr2,
           Wc1, bc1, Wc2, bc2, Wc3, bc3):
    src2d = edge_index[0].reshape(E // _CHUNK, _CHUNK)
    dst2d = edge_index[1].reshape(E // _CHUNK, _CHUNK)

    aggp, degp = _seg_sum_deg(x, src2d, dst2d)
    h1 = _tc_layer1(aggp, degp, x, Wl1, bl1.reshape(1, H), Wr1)
    aggp2, = _seg_sum(h1, src2d, dst2d)
    ps, pd = _tc_layer2(aggp2, degp, h1, Wl2, bl2.reshape(1, H), Wr2,
                        Wc1[:, :H], Wc1[:, H:2 * H])
    a, b = _edge_gather(ps, pd, src2d, dst2d)
    out = _tc_edge_mlp(a, b, edge_attr, Wc1[:, 2 * H:], bc1.reshape(1, H),
                       Wc2, bc2.reshape(1, 64), Wc3, bc3.reshape(1, 1))
    return out.reshape(-1)
